# baseline (speedup 1.0000x reference)
import functools

import jax
import jax.numpy as jnp
from jax import lax
from jax.experimental import pallas as pl
from jax.experimental.pallas import tpu as pltpu
from jax.experimental.pallas import tpu_sc as plsc

N = 10000
NPAD = 10240
E = 160000
EPAD = 163840
D = 256
HH = 128
NG = 64

f32 = jnp.float32
i32 = jnp.int32

@functools.cache
def _mesh():
    return plsc.VectorSubcoreMesh(core_axis_name="c", subcore_axis_name="s")


_SC_PARAMS = pltpu.CompilerParams(needs_layout_passes=False)


def _rsqrt_newton(v):
    bits = plsc.bitcast(v, i32)
    y = plsc.bitcast(jnp.int32(0x5F3759DF) - lax.shift_right_arithmetic(bits, 1), f32)
    for _ in range(4):
        y = y * (1.5 - 0.5 * v * y * y)
    return y


def _prep1_body(src_h, dst_h, px_h, py_h, pz_h, sig_h,
                ew_h, degp_h,
                pxv, pyv, pzv, srcb, dstb, ewb, degb, sigv, stgsem):
    c = lax.axis_index("c")
    s = lax.axis_index("s")
    w = s * 2 + c
    epw = EPAD // 32
    base = w * epw
    cps = [(px_h, pxv), (py_h, pyv), (pz_h, pzv), (sig_h, sigv),
           (src_h.at[pl.ds(base, epw)], srcb),
           (dst_h.at[pl.ds(base, epw)], dstb)]
    for src, dst in cps:
        pltpu.async_copy(src, dst, stgsem)
    for src, dst in cps:
        pltpu.make_async_copy(src, dst, stgsem).wait()

    def zero(i, _):
        degb[pl.ds(i * 16, 16)] = jnp.zeros((16,), f32)
        return 0
    lax.fori_loop(0, NPAD // 16, zero, 0)

    sig = sigv[...]

    def step(i, _):
        sv = srcb[pl.ds(i * 16, 16)]
        dv = dstb[pl.ds(i * 16, 16)]
        ax = plsc.load_gather(pxv, [sv])
        ay = plsc.load_gather(pyv, [sv])
        az = plsc.load_gather(pzv, [sv])
        bx = plsc.load_gather(pxv, [dv])
        by = plsc.load_gather(pyv, [dv])
        bz = plsc.load_gather(pzv, [dv])
        dx = ax - bx
        dy = ay - by
        dz = az - bz
        dist2 = dx * dx + dy * dy + dz * dz
        ew = jnp.exp(-(sig * dist2))
        ewb[pl.ds(i * 16, 16)] = ew
        plsc.addupdate_scatter(degb, [dv], ew)
        return 0
    lax.fori_loop(0, epw // 16, step, 0)

    pltpu.sync_copy(ewb, ew_h.at[pl.ds(base, epw)])
    pltpu.sync_copy(degb, degp_h.at[pl.ds(w * NPAD, NPAD)])


@functools.cache
def _prep1():
    return pl.kernel(
    _prep1_body,
    out_type=[jax.ShapeDtypeStruct((EPAD,), f32),
              jax.ShapeDtypeStruct((32 * NPAD,), f32)],
    mesh=_mesh(),
    compiler_params=_SC_PARAMS,
    scratch_types=[pltpu.VMEM((NPAD,), f32),
                   pltpu.VMEM((NPAD,), f32),
                   pltpu.VMEM((NPAD,), f32),
                   pltpu.VMEM((EPAD // 32,), i32),
                   pltpu.VMEM((EPAD // 32,), i32),
                   pltpu.VMEM((EPAD // 32,), f32),
                   pltpu.VMEM((NPAD,), f32),
                   pltpu.VMEM((16,), f32),
                   pltpu.SemaphoreType.DMA],
    )


def _prep2_body(degp_h, dis_h, accb, tmpb, sem):
    c = lax.axis_index("c")
    s = lax.axis_index("s")
    w = s * 2 + c
    npw = NPAD // 32
    base = w * npw

    for t in range(32):
        pltpu.async_copy(degp_h.at[pl.ds(t * NPAD + base, npw)],
                         tmpb.at[pl.ds(t * npw, npw)], sem)
    for t in range(32):
        pltpu.make_async_copy(degp_h.at[pl.ds(0, npw)],
                              tmpb.at[pl.ds(t * npw, npw)], sem).wait()

    def fin(i, _):
        acc = tmpb[pl.ds(i * 16, 16)]

        def add(t, a):
            return a + tmpb[pl.ds(t * npw + i * 16, 16)]
        acc = lax.fori_loop(1, 32, add, acc)
        accb[pl.ds(i * 16, 16)] = _rsqrt_newton(acc + 1.0)
        return 0
    lax.fori_loop(0, npw // 16, fin, 0)
    pltpu.sync_copy(accb, dis_h.at[pl.ds(base, npw)])


@functools.cache
def _prep2():
    return pl.kernel(
    _prep2_body,
    out_type=jax.ShapeDtypeStruct((NPAD,), f32),
    mesh=_mesh(),
    compiler_params=_SC_PARAMS,
    scratch_types=[pltpu.VMEM((NPAD // 32,), f32),
                   pltpu.VMEM((NPAD,), f32),
                   pltpu.SemaphoreType.DMA],
    )


def _prep3_body(src_h, dst_h, ew_h, dis_h, norm_h,
                disv, srcb, dstb, ewb, normb):
    c = lax.axis_index("c")
    s = lax.axis_index("s")
    w = s * 2 + c
    epw = EPAD // 32
    base = w * epw
    pltpu.sync_copy(dis_h, disv)
    pltpu.sync_copy(src_h.at[pl.ds(base, epw)], srcb)
    pltpu.sync_copy(dst_h.at[pl.ds(base, epw)], dstb)
    pltpu.sync_copy(ew_h.at[pl.ds(base, epw)], ewb)

    def step(i, _):
        sv = srcb[pl.ds(i * 16, 16)]
        dv = dstb[pl.ds(i * 16, 16)]
        ds_ = plsc.load_gather(disv, [sv])
        dd_ = plsc.load_gather(disv, [dv])
        normb[pl.ds(i * 16, 16)] = ds_ * ewb[pl.ds(i * 16, 16)] * dd_
        return 0
    lax.fori_loop(0, epw // 16, step, 0)
    pltpu.sync_copy(normb, norm_h.at[pl.ds(base, epw)])


@functools.cache
def _prep3():
    return pl.kernel(
    _prep3_body,
    out_type=jax.ShapeDtypeStruct((EPAD,), f32),
    mesh=_mesh(),
    compiler_params=_SC_PARAMS,
    scratch_types=[pltpu.VMEM((NPAD,), f32),
                   pltpu.VMEM((EPAD // 32,), i32),
                   pltpu.VMEM((EPAD // 32,), i32),
                   pltpu.VMEM((EPAD // 32,), f32),
                   pltpu.VMEM((EPAD // 32,), f32)],
    )


_CHUNK = 64
_EPT = EPAD // 16
_RPT = NPAD // 16
_NCH = _EPT // _CHUNK
_NB = 4


def _conv_body(h_h, init_h, src_h, dst_h, norm_h, msg_h,
               acc, normb,
               srcc0, srcc1, srcc2, srcc3, dstc0, dstc1, dstc2, dstc3,
               idx0, idx1, idx2, idx3, dsc0, dsc1, dsc2, dsc3,
               rows0, rows1, rows2, rows3,
               semg0, semg1, semg2, semg3, semm0, semm1, semm2, semm3,
               sems0, sems1, sems2, sems3):
    c = lax.axis_index("c")
    s = lax.axis_index("s")
    row0 = s * _RPT
    pltpu.sync_copy(init_h.at[pl.ds(c * NPAD + row0, _RPT)], acc.at[pl.ds(row0, _RPT)])
    coff = c * NPAD
    e0 = s * _EPT
    pltpu.sync_copy(norm_h.at[pl.ds(e0, _EPT)], normb)
    plsc.subcore_barrier()

    srccs = (srcc0, srcc1, srcc2, srcc3)
    dstcs = (dstc0, dstc1, dstc2, dstc3)
    idxs = (idx0, idx1, idx2, idx3)
    dscs = (dsc0, dsc1, dsc2, dsc3)
    rowss = (rows0, rows1, rows2, rows3)
    semgs = (semg0, semg1, semg2, semg3)
    semms = (semm0, semm1, semm2, semm3)
    semss = (sems0, sems1, sems2, sems3)

    def fire_meta(k, b):
        base = e0 + k * _CHUNK
        pltpu.async_copy(src_h.at[pl.ds(base, _CHUNK)], srccs[b], semms[b])
        pltpu.async_copy(dst_h.at[pl.ds(base, _CHUNK)], dstcs[b], semms[b])

    def drain_meta(b):
        pltpu.make_async_copy(src_h.at[pl.ds(0, _CHUNK)], srccs[b], semms[b]).wait()
        pltpu.make_async_copy(dst_h.at[pl.ds(0, _CHUNK)], dstcs[b], semms[b]).wait()

    def mkidx(b):
        def mk(j, _):
            sl = pl.ds(j * 16, 16)
            idxs[b][sl] = srccs[b][sl] + coff
            return 0
        lax.fori_loop(0, _CHUNK // 16, mk, 0, unroll=True)

    def fire_gather(b):
        pltpu.async_copy(h_h.at[idxs[b]], rowss[b], semgs[b])

    def drain_scatter(b):
        pltpu.make_async_copy(rowss[b], acc.at[dscs[b]], semss[b]).wait()

    def step1(k, b2):
        @pl.when(k + 2 < _NCH)
        def _():
            drain_meta(b2)
            mkidx(b2)

            @pl.when(k >= 2)
            def _():
                drain_scatter(b2)
            fire_gather(b2)

    def process(k, b):
        rows = rowss[b]
        pltpu.make_async_copy(h_h.at[idxs[b]], rowss[b], semgs[b]).wait()
        base = k * _CHUNK

        def scale(r, _):
            nb = plsc.load_gather(normb, [jnp.full((16,), base + r, i32)])
            for j in range(HH // 16):
                rows[r, pl.ds(j * 16, 16)] = rows[r, pl.ds(j * 16, 16)] * nb
            return 0
        lax.fori_loop(0, _CHUNK, scale, 0, unroll=8)

        def cpdst(j, _):
            sl = pl.ds(j * 16, 16)
            dscs[b][sl] = dstcs[b][sl]
            return 0
        lax.fori_loop(0, _CHUNK // 16, cpdst, 0, unroll=True)
        pltpu.async_copy(rows, acc.at[dscs[b]], semss[b], add=True)

    def whole_step(k, b, b2, bm):
        step1(k, b2)
        process(k, b)

        @pl.when(k + 4 < _NCH)
        def _():
            fire_meta(k + 4, bm)

    fire_meta(0, 0)
    fire_meta(1, 1)
    fire_meta(2, 2)
    fire_meta(3, 3)
    drain_meta(0)
    mkidx(0)
    fire_gather(0)
    drain_meta(1)
    mkidx(1)
    fire_gather(1)

    def body(kk, _):
        for j in range(_NB):
            k = kk * _NB + j
            whole_step(k, j, (j + 2) % _NB, j)
        return 0

    lax.fori_loop(0, _NCH // _NB, body, 0)
    drain_scatter((_NCH - 4) % _NB)
    drain_scatter((_NCH - 3) % _NB)
    drain_scatter((_NCH - 2) % _NB)
    drain_scatter((_NCH - 1) % _NB)
    plsc.subcore_barrier()
    pltpu.sync_copy(acc.at[pl.ds(row0, _RPT)], msg_h.at[pl.ds(c * NPAD + row0, _RPT)])


@functools.cache
def _conv():
    return pl.kernel(
    _conv_body,
    out_type=jax.ShapeDtypeStruct((2 * NPAD, HH), f32),
    mesh=_mesh(),
    compiler_params=_SC_PARAMS,
    scratch_types=[pltpu.VMEM_SHARED((NPAD, HH), f32),
                   pltpu.VMEM((_EPT,), f32)]
                  + [pltpu.VMEM((_CHUNK,), i32)] * 16
                  + [pltpu.VMEM((_CHUNK, HH), f32)] * 4
                  + [pltpu.SemaphoreType.DMA] * 12,
    )


def _matmul_init_body(x_ref, w_ref, b_ref, dis_ref, h_ref, init_ref):
    x = x_ref[...]
    h = jnp.dot(x, w_ref[...], preferred_element_type=f32)
    d2 = dis_ref[...] * dis_ref[...]
    init = h * d2 + b_ref[...]
    h_ref[0] = h[:, :HH]
    h_ref[1] = h[:, HH:]
    init_ref[0] = init[:, :HH]
    init_ref[1] = init[:, HH:]


def _matmul_init(x, w, b, dis):
    return pl.pallas_call(
        _matmul_init_body,
        out_shape=[jax.ShapeDtypeStruct((2, NPAD, HH), f32),
                   jax.ShapeDtypeStruct((2, NPAD, HH), f32)],
    )(x, w, b.reshape(1, D), dis.reshape(NPAD, 1))


def _bn_half(m, g, be, mask):
    mu = jnp.sum(m * mask, axis=0, keepdims=True) * (1.0 / N)
    dmu = m - mu
    var = jnp.sum(dmu * dmu * mask, axis=0, keepdims=True) * (1.0 / N)
    return g * dmu * jax.lax.rsqrt(var + 1e-5) + be


def _bn_relu_nores_body(msg_ref, g_ref, be_ref, y_ref):
    mask = (lax.broadcasted_iota(i32, (NPAD, 1), 0) < N).astype(f32)
    for half in range(2):
        m = msg_ref[half]
        g = g_ref[0:1, half * HH:half * HH + HH]
        be = be_ref[0:1, half * HH:half * HH + HH]
        y = _bn_half(m, g, be, mask)
        y_ref[:, half * HH:half * HH + HH] = jnp.maximum(y, 0.0)


def _bn_relu_res_body(msg_ref, g_ref, be_ref, res_ref, y_ref):
    mask = (lax.broadcasted_iota(i32, (NPAD, 1), 0) < N).astype(f32)
    for half in range(2):
        m = msg_ref[half]
        g = g_ref[0:1, half * HH:half * HH + HH]
        be = be_ref[0:1, half * HH:half * HH + HH]
        y = _bn_half(m, g, be, mask)
        y = y + res_ref[:, half * HH:half * HH + HH]
        y_ref[:, half * HH:half * HH + HH] = jnp.maximum(y, 0.0)


def _bn_relu(msg2, g, be, res=None):
    args = [msg2, g.reshape(1, D), be.reshape(1, D)]
    body = _bn_relu_nores_body
    if res is not None:
        args.append(res)
        body = _bn_relu_res_body
    return pl.pallas_call(
        body,
        out_shape=jax.ShapeDtypeStruct((NPAD, D), f32),
    )(*args)


def _final_body(msg_ref, g_ref, be_ref, res_ref, batch_ref,
                lw0_ref, lb0_ref, lw1_ref, lb1_ref, out_ref):
    mask = (lax.broadcasted_iota(i32, (NPAD, 1), 0) < N).astype(f32)
    halves = []
    for half in range(2):
        m = msg_ref[half]
        g = g_ref[0:1, half * HH:half * HH + HH]
        be = be_ref[0:1, half * HH:half * HH + HH]
        y = _bn_half(m, g, be, mask)
        y = y + res_ref[:, half * HH:half * HH + HH]
        halves.append(jnp.maximum(y, 0.0))
    y4 = jnp.concatenate(halves, axis=1)
    seg = lax.broadcasted_iota(i32, (NG, NPAD), 0)
    onehot = (seg == batch_ref[...]).astype(f32)
    pooled = jnp.dot(onehot, y4, preferred_element_type=f32,
                     precision=jax.lax.Precision.HIGHEST)
    cnt = jnp.sum(onehot, axis=1, keepdims=True)
    mean = pooled / jnp.maximum(cnt, 1.0)
    z = jnp.maximum(jnp.dot(mean, lw0_ref[...], preferred_element_type=f32)
                    + lb0_ref[...], 0.0)
    out_ref[...] = (jnp.dot(z, lw1_ref[...], preferred_element_type=f32)
                    + lb1_ref[...])


def _final(msg2, g, be, res, batch2, lw0, lb0, lw1, lb1):
    return pl.pallas_call(
        _final_body,
        out_shape=jax.ShapeDtypeStruct((NG, lw1.shape[1]), f32),
    )(msg2, g.reshape(1, D), be.reshape(1, D), res, batch2,
      lw0, lb0.reshape(1, -1), lw1, lb1.reshape(1, -1))


def kernel(x, edge_index, pos, batch, inv_sigma,
           W1_0, b1_0, g1_0, be1_0, W2_0, b2_0, g2_0, be2_0,
           W1_1, b1_1, g1_1, be1_1, W2_1, b2_1, g2_1, be2_1,
           lW0, lb0, lW1, lb1):
    npad = NPAD - N
    epad = EPAD - E
    xp = jnp.pad(x, ((0, npad), (0, 0)))
    posp = jnp.pad(pos, ((0, npad), (0, 0)))
    px = posp[:, 0] + 0.0
    py = posp[:, 1] + 0.0
    pz = posp[:, 2] + 0.0
    padi = (N + (jnp.arange(epad, dtype=i32) % npad)).astype(i32)
    srcp = jnp.concatenate([edge_index[0].astype(i32), padi])
    dstp = jnp.concatenate([edge_index[1].astype(i32), padi])
    batchp = jnp.pad(batch.astype(i32), (0, npad), constant_values=NG)
    batch2 = batchp.reshape(1, NPAD)
    sigb = jnp.broadcast_to(inv_sigma.astype(f32), (16,))

    ew, degp = _prep1()(srcp, dstp, px, py, pz, sigb)
    dis = _prep2()(degp)
    norm = _prep3()(srcp, dstp, ew, dis)

    def conv(h_in, w, b):
        h2, init2 = _matmul_init(h_in, w, b, dis)
        m = _conv()(h2.reshape(2 * NPAD, HH), init2.reshape(2 * NPAD, HH),
                    srcp, dstp, norm)
        return m.reshape(2, NPAD, HH)

    m = conv(xp, W1_0, b1_0)
    y = _bn_relu(m, g1_0, be1_0)
    m = conv(y, W2_0, b2_0)
    y2 = _bn_relu(m, g2_0, be2_0, res=xp)
    m = conv(y2, W1_1, b1_1)
    y = _bn_relu(m, g1_1, be1_1)
    m = conv(y, W2_1, b2_1)
    return _final(m, g2_1, be2_1, y2, batch2, lW0, lb0, lW1, lb1)


# speedup vs baseline: 1.0071x; 1.0071x over previous
import functools

import jax
import jax.numpy as jnp
from jax import lax
from jax.experimental import pallas as pl
from jax.experimental.pallas import tpu as pltpu
from jax.experimental.pallas import tpu_sc as plsc

N = 10000
NPAD = 10240
E = 160000
EPAD = 163840
D = 256
HH = 128
NG = 64

f32 = jnp.float32
i32 = jnp.int32

@functools.cache
def _mesh():
    return plsc.VectorSubcoreMesh(core_axis_name="c", subcore_axis_name="s")


_SC_PARAMS = pltpu.CompilerParams(needs_layout_passes=False)


def _rsqrt_newton(v):
    bits = plsc.bitcast(v, i32)
    y = plsc.bitcast(jnp.int32(0x5F3759DF) - lax.shift_right_arithmetic(bits, 1), f32)
    for _ in range(4):
        y = y * (1.5 - 0.5 * v * y * y)
    return y


def _prep1_body(src_h, dst_h, px_h, py_h, pz_h, sig_h,
                ew_h, degp_h,
                pxv, pyv, pzv, srcb, dstb, ewb, degb, sigv, stgsem):
    c = lax.axis_index("c")
    s = lax.axis_index("s")
    w = s * 2 + c
    epw = EPAD // 32
    base = w * epw
    cps = [(px_h, pxv), (py_h, pyv), (pz_h, pzv), (sig_h, sigv),
           (src_h.at[pl.ds(base, epw)], srcb),
           (dst_h.at[pl.ds(base, epw)], dstb)]
    for src, dst in cps:
        pltpu.async_copy(src, dst, stgsem)
    for src, dst in cps:
        pltpu.make_async_copy(src, dst, stgsem).wait()

    def zero(i, _):
        degb[pl.ds(i * 16, 16)] = jnp.zeros((16,), f32)
        return 0
    lax.fori_loop(0, NPAD // 16, zero, 0)

    sig = sigv[...]

    def step(i, _):
        sv = srcb[pl.ds(i * 16, 16)]
        dv = dstb[pl.ds(i * 16, 16)]
        ax = plsc.load_gather(pxv, [sv])
        ay = plsc.load_gather(pyv, [sv])
        az = plsc.load_gather(pzv, [sv])
        bx = plsc.load_gather(pxv, [dv])
        by = plsc.load_gather(pyv, [dv])
        bz = plsc.load_gather(pzv, [dv])
        dx = ax - bx
        dy = ay - by
        dz = az - bz
        dist2 = dx * dx + dy * dy + dz * dz
        ew = jnp.exp(-(sig * dist2))
        ewb[pl.ds(i * 16, 16)] = ew
        plsc.addupdate_scatter(degb, [dv], ew)
        return 0
    lax.fori_loop(0, epw // 16, step, 0)

    pltpu.sync_copy(ewb, ew_h.at[pl.ds(base, epw)])
    pltpu.sync_copy(degb, degp_h.at[pl.ds(w * NPAD, NPAD)])


@functools.cache
def _prep1():
    return pl.kernel(
    _prep1_body,
    out_type=[jax.ShapeDtypeStruct((EPAD,), f32),
              jax.ShapeDtypeStruct((32 * NPAD,), f32)],
    mesh=_mesh(),
    compiler_params=_SC_PARAMS,
    scratch_types=[pltpu.VMEM((NPAD,), f32),
                   pltpu.VMEM((NPAD,), f32),
                   pltpu.VMEM((NPAD,), f32),
                   pltpu.VMEM((EPAD // 32,), i32),
                   pltpu.VMEM((EPAD // 32,), i32),
                   pltpu.VMEM((EPAD // 32,), f32),
                   pltpu.VMEM((NPAD,), f32),
                   pltpu.VMEM((16,), f32),
                   pltpu.SemaphoreType.DMA],
    )


def _prep2_body(degp_h, dis_h, accb, tmpb, sem):
    c = lax.axis_index("c")
    s = lax.axis_index("s")
    w = s * 2 + c
    npw = NPAD // 32
    base = w * npw

    for t in range(32):
        pltpu.async_copy(degp_h.at[pl.ds(t * NPAD + base, npw)],
                         tmpb.at[pl.ds(t * npw, npw)], sem)
    for t in range(32):
        pltpu.make_async_copy(degp_h.at[pl.ds(0, npw)],
                              tmpb.at[pl.ds(t * npw, npw)], sem).wait()

    def fin(i, _):
        acc = tmpb[pl.ds(i * 16, 16)]

        def add(t, a):
            return a + tmpb[pl.ds(t * npw + i * 16, 16)]
        acc = lax.fori_loop(1, 32, add, acc)
        accb[pl.ds(i * 16, 16)] = _rsqrt_newton(acc + 1.0)
        return 0
    lax.fori_loop(0, npw // 16, fin, 0)
    pltpu.sync_copy(accb, dis_h.at[pl.ds(base, npw)])


@functools.cache
def _prep2():
    return pl.kernel(
    _prep2_body,
    out_type=jax.ShapeDtypeStruct((NPAD,), f32),
    mesh=_mesh(),
    compiler_params=_SC_PARAMS,
    scratch_types=[pltpu.VMEM((NPAD // 32,), f32),
                   pltpu.VMEM((NPAD,), f32),
                   pltpu.SemaphoreType.DMA],
    )


def _prep3_body(src_h, dst_h, ew_h, dis_h, norm_h,
                disv, srcb, dstb, ewb, normb):
    c = lax.axis_index("c")
    s = lax.axis_index("s")
    w = s * 2 + c
    epw = EPAD // 32
    base = w * epw
    pltpu.sync_copy(dis_h, disv)
    pltpu.sync_copy(src_h.at[pl.ds(base, epw)], srcb)
    pltpu.sync_copy(dst_h.at[pl.ds(base, epw)], dstb)
    pltpu.sync_copy(ew_h.at[pl.ds(base, epw)], ewb)

    def step(i, _):
        sv = srcb[pl.ds(i * 16, 16)]
        dv = dstb[pl.ds(i * 16, 16)]
        ds_ = plsc.load_gather(disv, [sv])
        dd_ = plsc.load_gather(disv, [dv])
        normb[pl.ds(i * 16, 16)] = ds_ * ewb[pl.ds(i * 16, 16)] * dd_
        return 0
    lax.fori_loop(0, epw // 16, step, 0)
    pltpu.sync_copy(normb, norm_h.at[pl.ds(base, epw)])


@functools.cache
def _prep3():
    return pl.kernel(
    _prep3_body,
    out_type=jax.ShapeDtypeStruct((EPAD,), f32),
    mesh=_mesh(),
    compiler_params=_SC_PARAMS,
    scratch_types=[pltpu.VMEM((NPAD,), f32),
                   pltpu.VMEM((EPAD // 32,), i32),
                   pltpu.VMEM((EPAD // 32,), i32),
                   pltpu.VMEM((EPAD // 32,), f32),
                   pltpu.VMEM((EPAD // 32,), f32)],
    )


_CHUNK = 64
_EPT = EPAD // 16
_RPT = NPAD // 16
_NCH = _EPT // _CHUNK
_NB = 4


def _conv_body(h_h, init_h, src_h, dst_h, norm_h, msg_h,
               acc, normb,
               srcc0, srcc1, srcc2, srcc3, dstc0, dstc1, dstc2, dstc3,
               idx0, idx1, idx2, idx3, dsc0, dsc1, dsc2, dsc3,
               rows0, rows1, rows2, rows3,
               semg0, semg1, semg2, semg3, semm0, semm1, semm2, semm3,
               sems0, sems1, sems2, sems3):
    c = lax.axis_index("c")
    s = lax.axis_index("s")
    row0 = s * _RPT
    pltpu.sync_copy(init_h.at[pl.ds(c * NPAD + row0, _RPT)], acc.at[pl.ds(row0, _RPT)])
    coff = c * NPAD
    e0 = s * _EPT
    pltpu.sync_copy(norm_h.at[pl.ds(e0, _EPT)], normb)
    plsc.subcore_barrier()

    srccs = (srcc0, srcc1, srcc2, srcc3)
    dstcs = (dstc0, dstc1, dstc2, dstc3)
    idxs = (idx0, idx1, idx2, idx3)
    dscs = (dsc0, dsc1, dsc2, dsc3)
    rowss = (rows0, rows1, rows2, rows3)
    semgs = (semg0, semg1, semg2, semg3)
    semms = (semm0, semm1, semm2, semm3)
    semss = (sems0, sems1, sems2, sems3)

    def fire_meta(k, b):
        base = e0 + k * _CHUNK
        pltpu.async_copy(src_h.at[pl.ds(base, _CHUNK)], srccs[b], semms[b])
        pltpu.async_copy(dst_h.at[pl.ds(base, _CHUNK)], dstcs[b], semms[b])

    def drain_meta(b):
        pltpu.make_async_copy(src_h.at[pl.ds(0, _CHUNK)], srccs[b], semms[b]).wait()
        pltpu.make_async_copy(dst_h.at[pl.ds(0, _CHUNK)], dstcs[b], semms[b]).wait()

    def mkidx(b):
        def mk(j, _):
            sl = pl.ds(j * 16, 16)
            idxs[b][sl] = srccs[b][sl] + coff
            return 0
        lax.fori_loop(0, _CHUNK // 16, mk, 0, unroll=True)

    def fire_gather(b):
        pltpu.async_copy(h_h.at[idxs[b]], rowss[b], semgs[b])

    def drain_scatter(b):
        pltpu.make_async_copy(rowss[b], acc.at[dscs[b]], semss[b]).wait()

    def step1(k, b2):
        @pl.when(k + 2 < _NCH)
        def _():
            drain_meta(b2)
            mkidx(b2)

            @pl.when(k >= 2)
            def _():
                drain_scatter(b2)
            fire_gather(b2)

    def process(k, b):
        rows = rowss[b]
        pltpu.make_async_copy(h_h.at[idxs[b]], rowss[b], semgs[b]).wait()
        base = k * _CHUNK

        def scale(r, _):
            nb = plsc.load_gather(normb, [jnp.full((16,), base + r, i32)])
            for j in range(HH // 16):
                rows[r, pl.ds(j * 16, 16)] = rows[r, pl.ds(j * 16, 16)] * nb
            return 0
        lax.fori_loop(0, _CHUNK, scale, 0, unroll=4)

        def cpdst(j, _):
            sl = pl.ds(j * 16, 16)
            dscs[b][sl] = dstcs[b][sl]
            return 0
        lax.fori_loop(0, _CHUNK // 16, cpdst, 0, unroll=True)
        pltpu.async_copy(rows, acc.at[dscs[b]], semss[b], add=True)

    def whole_step(k, b, b2, bm):
        step1(k, b2)
        process(k, b)

        @pl.when(k + 4 < _NCH)
        def _():
            fire_meta(k + 4, bm)

    fire_meta(0, 0)
    fire_meta(1, 1)
    fire_meta(2, 2)
    fire_meta(3, 3)
    drain_meta(0)
    mkidx(0)
    fire_gather(0)
    drain_meta(1)
    mkidx(1)
    fire_gather(1)

    def body(kk, _):
        for j in range(_NB):
            k = kk * _NB + j
            whole_step(k, j, (j + 2) % _NB, j)
        return 0

    lax.fori_loop(0, _NCH // _NB, body, 0)
    drain_scatter((_NCH - 4) % _NB)
    drain_scatter((_NCH - 3) % _NB)
    drain_scatter((_NCH - 2) % _NB)
    drain_scatter((_NCH - 1) % _NB)
    plsc.subcore_barrier()
    pltpu.sync_copy(acc.at[pl.ds(row0, _RPT)], msg_h.at[pl.ds(c * NPAD + row0, _RPT)])


@functools.cache
def _conv():
    return pl.kernel(
    _conv_body,
    out_type=jax.ShapeDtypeStruct((2 * NPAD, HH), f32),
    mesh=_mesh(),
    compiler_params=_SC_PARAMS,
    scratch_types=[pltpu.VMEM_SHARED((NPAD, HH), f32),
                   pltpu.VMEM((_EPT,), f32)]
                  + [pltpu.VMEM((_CHUNK,), i32)] * 16
                  + [pltpu.VMEM((_CHUNK, HH), f32)] * 4
                  + [pltpu.SemaphoreType.DMA] * 12,
    )


def _matmul_init_body(x_ref, w_ref, b_ref, dis_ref, h_ref, init_ref):
    x = x_ref[...]
    h = jnp.dot(x, w_ref[...], preferred_element_type=f32)
    d2 = dis_ref[...] * dis_ref[...]
    init = h * d2 + b_ref[...]
    h_ref[0] = h[:, :HH]
    h_ref[1] = h[:, HH:]
    init_ref[0] = init[:, :HH]
    init_ref[1] = init[:, HH:]


def _matmul_init(x, w, b, dis):
    return pl.pallas_call(
        _matmul_init_body,
        out_shape=[jax.ShapeDtypeStruct((2, NPAD, HH), f32),
                   jax.ShapeDtypeStruct((2, NPAD, HH), f32)],
    )(x, w, b.reshape(1, D), dis.reshape(NPAD, 1))


def _bn_half(m, g, be, mask):
    mu = jnp.sum(m * mask, axis=0, keepdims=True) * (1.0 / N)
    dmu = m - mu
    var = jnp.sum(dmu * dmu * mask, axis=0, keepdims=True) * (1.0 / N)
    return g * dmu * jax.lax.rsqrt(var + 1e-5) + be


def _bn_relu_nores_body(msg_ref, g_ref, be_ref, y_ref):
    mask = (lax.broadcasted_iota(i32, (NPAD, 1), 0) < N).astype(f32)
    for half in range(2):
        m = msg_ref[half]
        g = g_ref[0:1, half * HH:half * HH + HH]
        be = be_ref[0:1, half * HH:half * HH + HH]
        y = _bn_half(m, g, be, mask)
        y_ref[:, half * HH:half * HH + HH] = jnp.maximum(y, 0.0)


def _bn_relu_res_body(msg_ref, g_ref, be_ref, res_ref, y_ref):
    mask = (lax.broadcasted_iota(i32, (NPAD, 1), 0) < N).astype(f32)
    for half in range(2):
        m = msg_ref[half]
        g = g_ref[0:1, half * HH:half * HH + HH]
        be = be_ref[0:1, half * HH:half * HH + HH]
        y = _bn_half(m, g, be, mask)
        y = y + res_ref[:, half * HH:half * HH + HH]
        y_ref[:, half * HH:half * HH + HH] = jnp.maximum(y, 0.0)


def _bn_relu(msg2, g, be, res=None):
    args = [msg2, g.reshape(1, D), be.reshape(1, D)]
    body = _bn_relu_nores_body
    if res is not None:
        args.append(res)
        body = _bn_relu_res_body
    return pl.pallas_call(
        body,
        out_shape=jax.ShapeDtypeStruct((NPAD, D), f32),
    )(*args)


def _final_body(msg_ref, g_ref, be_ref, res_ref, batch_ref,
                lw0_ref, lb0_ref, lw1_ref, lb1_ref, out_ref):
    mask = (lax.broadcasted_iota(i32, (NPAD, 1), 0) < N).astype(f32)
    halves = []
    for half in range(2):
        m = msg_ref[half]
        g = g_ref[0:1, half * HH:half * HH + HH]
        be = be_ref[0:1, half * HH:half * HH + HH]
        y = _bn_half(m, g, be, mask)
        y = y + res_ref[:, half * HH:half * HH + HH]
        halves.append(jnp.maximum(y, 0.0))
    y4 = jnp.concatenate(halves, axis=1)
    seg = lax.broadcasted_iota(i32, (NG, NPAD), 0)
    onehot = (seg == batch_ref[...]).astype(f32)
    pooled = jnp.dot(onehot, y4, preferred_element_type=f32,
                     precision=jax.lax.Precision.HIGHEST)
    cnt = jnp.sum(onehot, axis=1, keepdims=True)
    mean = pooled / jnp.maximum(cnt, 1.0)
    z = jnp.maximum(jnp.dot(mean, lw0_ref[...], preferred_element_type=f32)
                    + lb0_ref[...], 0.0)
    out_ref[...] = (jnp.dot(z, lw1_ref[...], preferred_element_type=f32)
                    + lb1_ref[...])


def _final(msg2, g, be, res, batch2, lw0, lb0, lw1, lb1):
    return pl.pallas_call(
        _final_body,
        out_shape=jax.ShapeDtypeStruct((NG, lw1.shape[1]), f32),
    )(msg2, g.reshape(1, D), be.reshape(1, D), res, batch2,
      lw0, lb0.reshape(1, -1), lw1, lb1.reshape(1, -1))


def kernel(x, edge_index, pos, batch, inv_sigma,
           W1_0, b1_0, g1_0, be1_0, W2_0, b2_0, g2_0, be2_0,
           W1_1, b1_1, g1_1, be1_1, W2_1, b2_1, g2_1, be2_1,
           lW0, lb0, lW1, lb1):
    npad = NPAD - N
    epad = EPAD - E
    xp = jnp.pad(x, ((0, npad), (0, 0)))
    posp = jnp.pad(pos, ((0, npad), (0, 0)))
    px = posp[:, 0] + 0.0
    py = posp[:, 1] + 0.0
    pz = posp[:, 2] + 0.0
    padi = (N + (jnp.arange(epad, dtype=i32) % npad)).astype(i32)
    srcp = jnp.concatenate([edge_index[0].astype(i32), padi])
    dstp = jnp.concatenate([edge_index[1].astype(i32), padi])
    batchp = jnp.pad(batch.astype(i32), (0, npad), constant_values=NG)
    batch2 = batchp.reshape(1, NPAD)
    sigb = jnp.broadcast_to(inv_sigma.astype(f32), (16,))

    ew, degp = _prep1()(srcp, dstp, px, py, pz, sigb)
    dis = _prep2()(degp)
    norm = _prep3()(srcp, dstp, ew, dis)

    def conv(h_in, w, b):
        h2, init2 = _matmul_init(h_in, w, b, dis)
        m = _conv()(h2.reshape(2 * NPAD, HH), init2.reshape(2 * NPAD, HH),
                    srcp, dstp, norm)
        return m.reshape(2, NPAD, HH)

    m = conv(xp, W1_0, b1_0)
    y = _bn_relu(m, g1_0, be1_0)
    m = conv(y, W2_0, b2_0)
    y2 = _bn_relu(m, g2_0, be2_0, res=xp)
    m = conv(y2, W1_1, b1_1)
    y = _bn_relu(m, g1_1, be1_1)
    m = conv(y, W2_1, b2_1)
    return _final(m, g2_1, be2_1, y2, batch2, lW0, lb0, lW1, lb1)


# speedup vs baseline: 1.0344x; 1.0271x over previous
import functools

import jax
import jax.numpy as jnp
from jax import lax
from jax.experimental import pallas as pl
from jax.experimental.pallas import tpu as pltpu
from jax.experimental.pallas import tpu_sc as plsc

N = 10000
NPAD = 10240
E = 160000
EPAD = 163840
D = 256
HH = 128
NG = 64

f32 = jnp.float32
i32 = jnp.int32

@functools.cache
def _mesh():
    return plsc.VectorSubcoreMesh(core_axis_name="c", subcore_axis_name="s")


_SC_PARAMS = pltpu.CompilerParams(needs_layout_passes=False)


def _rsqrt_newton(v):
    bits = plsc.bitcast(v, i32)
    y = plsc.bitcast(jnp.int32(0x5F3759DF) - lax.shift_right_arithmetic(bits, 1), f32)
    for _ in range(4):
        y = y * (1.5 - 0.5 * v * y * y)
    return y


def _prep1_body(src_h, dst_h, px_h, py_h, pz_h, sig_h,
                ew_h, degp_h,
                pxv, pyv, pzv, srcb, dstb, ewb, degb, sigv, stgsem):
    c = lax.axis_index("c")
    s = lax.axis_index("s")
    w = s * 2 + c
    epw = EPAD // 32
    base = w * epw
    cps = [(px_h, pxv), (py_h, pyv), (pz_h, pzv), (sig_h, sigv),
           (src_h.at[pl.ds(base, epw)], srcb),
           (dst_h.at[pl.ds(base, epw)], dstb)]
    for src, dst in cps:
        pltpu.async_copy(src, dst, stgsem)
    for src, dst in cps:
        pltpu.make_async_copy(src, dst, stgsem).wait()

    def zero(i, _):
        degb[pl.ds(i * 16, 16)] = jnp.zeros((16,), f32)
        return 0
    lax.fori_loop(0, NPAD // 16, zero, 0)

    sig = sigv[...]

    def step(i, _):
        sv = srcb[pl.ds(i * 16, 16)]
        dv = dstb[pl.ds(i * 16, 16)]
        ax = plsc.load_gather(pxv, [sv])
        ay = plsc.load_gather(pyv, [sv])
        az = plsc.load_gather(pzv, [sv])
        bx = plsc.load_gather(pxv, [dv])
        by = plsc.load_gather(pyv, [dv])
        bz = plsc.load_gather(pzv, [dv])
        dx = ax - bx
        dy = ay - by
        dz = az - bz
        dist2 = dx * dx + dy * dy + dz * dz
        ew = jnp.exp(-(sig * dist2))
        ewb[pl.ds(i * 16, 16)] = ew
        plsc.addupdate_scatter(degb, [dv], ew)
        return 0
    lax.fori_loop(0, epw // 16, step, 0)

    pltpu.sync_copy(ewb, ew_h.at[pl.ds(base, epw)])
    pltpu.sync_copy(degb, degp_h.at[pl.ds(w * NPAD, NPAD)])


@functools.cache
def _prep1():
    return pl.kernel(
    _prep1_body,
    out_type=[jax.ShapeDtypeStruct((EPAD,), f32),
              jax.ShapeDtypeStruct((32 * NPAD,), f32)],
    mesh=_mesh(),
    compiler_params=_SC_PARAMS,
    scratch_types=[pltpu.VMEM((NPAD,), f32),
                   pltpu.VMEM((NPAD,), f32),
                   pltpu.VMEM((NPAD,), f32),
                   pltpu.VMEM((EPAD // 32,), i32),
                   pltpu.VMEM((EPAD // 32,), i32),
                   pltpu.VMEM((EPAD // 32,), f32),
                   pltpu.VMEM((NPAD,), f32),
                   pltpu.VMEM((16,), f32),
                   pltpu.SemaphoreType.DMA],
    )


def _prep2_body(degp_h, dis_h, accb, tmpb, sem):
    c = lax.axis_index("c")
    s = lax.axis_index("s")
    w = s * 2 + c
    npw = NPAD // 32
    base = w * npw

    for t in range(32):
        pltpu.async_copy(degp_h.at[pl.ds(t * NPAD + base, npw)],
                         tmpb.at[pl.ds(t * npw, npw)], sem)
    for t in range(32):
        pltpu.make_async_copy(degp_h.at[pl.ds(0, npw)],
                              tmpb.at[pl.ds(t * npw, npw)], sem).wait()

    def fin(i, _):
        acc = tmpb[pl.ds(i * 16, 16)]

        def add(t, a):
            return a + tmpb[pl.ds(t * npw + i * 16, 16)]
        acc = lax.fori_loop(1, 32, add, acc)
        accb[pl.ds(i * 16, 16)] = _rsqrt_newton(acc + 1.0)
        return 0
    lax.fori_loop(0, npw // 16, fin, 0)
    pltpu.sync_copy(accb, dis_h.at[pl.ds(base, npw)])


@functools.cache
def _prep2():
    return pl.kernel(
    _prep2_body,
    out_type=jax.ShapeDtypeStruct((NPAD,), f32),
    mesh=_mesh(),
    compiler_params=_SC_PARAMS,
    scratch_types=[pltpu.VMEM((NPAD // 32,), f32),
                   pltpu.VMEM((NPAD,), f32),
                   pltpu.SemaphoreType.DMA],
    )


def _prep3_body(src_h, dst_h, ew_h, dis_h, norm_h,
                disv, srcb, dstb, ewb, normb):
    c = lax.axis_index("c")
    s = lax.axis_index("s")
    w = s * 2 + c
    epw = EPAD // 32
    base = w * epw
    pltpu.sync_copy(dis_h, disv)
    pltpu.sync_copy(src_h.at[pl.ds(base, epw)], srcb)
    pltpu.sync_copy(dst_h.at[pl.ds(base, epw)], dstb)
    pltpu.sync_copy(ew_h.at[pl.ds(base, epw)], ewb)

    def step(i, _):
        sv = srcb[pl.ds(i * 16, 16)]
        dv = dstb[pl.ds(i * 16, 16)]
        ds_ = plsc.load_gather(disv, [sv])
        dd_ = plsc.load_gather(disv, [dv])
        normb[pl.ds(i * 16, 16)] = ds_ * ewb[pl.ds(i * 16, 16)] * dd_
        return 0
    lax.fori_loop(0, epw // 16, step, 0)
    pltpu.sync_copy(normb, norm_h.at[pl.ds(base, epw)])


@functools.cache
def _prep3():
    return pl.kernel(
    _prep3_body,
    out_type=jax.ShapeDtypeStruct((EPAD,), f32),
    mesh=_mesh(),
    compiler_params=_SC_PARAMS,
    scratch_types=[pltpu.VMEM((NPAD,), f32),
                   pltpu.VMEM((EPAD // 32,), i32),
                   pltpu.VMEM((EPAD // 32,), i32),
                   pltpu.VMEM((EPAD // 32,), f32),
                   pltpu.VMEM((EPAD // 32,), f32)],
    )


_CHUNK = 64
_EPT = EPAD // 16
_RPT = NPAD // 16
_NCH = _EPT // _CHUNK
_NB = 4


def _conv_body(h_h, init_h, src_h, dst_h, norm_h, msg_h,
               acc, normb,
               srcc0, srcc1, srcc2, srcc3, dstc0, dstc1, dstc2, dstc3,
               idx0, idx1, idx2, idx3, dsc0, dsc1, dsc2, dsc3,
               rows0, rows1, rows2, rows3,
               semg0, semg1, semg2, semg3, semm0, semm1, semm2, semm3,
               sems0, sems1, sems2, sems3):
    c = lax.axis_index("c")
    s = lax.axis_index("s")
    row0 = s * _RPT
    pltpu.sync_copy(init_h.at[pl.ds(c * NPAD + row0, _RPT)], acc.at[pl.ds(row0, _RPT)])
    coff = c * NPAD
    e0 = s * _EPT
    pltpu.sync_copy(norm_h.at[pl.ds(e0, _EPT)], normb)
    plsc.subcore_barrier()

    srccs = (srcc0, srcc1, srcc2, srcc3)
    dstcs = (dstc0, dstc1, dstc2, dstc3)
    idxs = (idx0, idx1, idx2, idx3)
    dscs = (dsc0, dsc1, dsc2, dsc3)
    rowss = (rows0, rows1, rows2, rows3)
    semgs = (semg0, semg1, semg2, semg3)
    semms = (semm0, semm1, semm2, semm3)
    semss = (sems0, sems1, sems2, sems3)

    def fire_meta(k, b):
        base = e0 + k * _CHUNK
        pltpu.async_copy(src_h.at[pl.ds(base, _CHUNK)], srccs[b], semms[b])
        pltpu.async_copy(dst_h.at[pl.ds(base, _CHUNK)], dstcs[b], semms[b])

    def drain_meta(b):
        pltpu.make_async_copy(src_h.at[pl.ds(0, _CHUNK)], srccs[b], semms[b]).wait()
        pltpu.make_async_copy(dst_h.at[pl.ds(0, _CHUNK)], dstcs[b], semms[b]).wait()

    def mkidx(b):
        def mk(j, _):
            sl = pl.ds(j * 16, 16)
            idxs[b][sl] = srccs[b][sl] + coff
            return 0
        lax.fori_loop(0, _CHUNK // 16, mk, 0, unroll=True)

    def fire_gather(b):
        pltpu.async_copy(h_h.at[idxs[b]], rowss[b], semgs[b])

    def drain_scatter(b):
        pltpu.make_async_copy(rowss[b], acc.at[dscs[b]], semss[b]).wait()

    def step1(k, b2):
        @pl.when(k + 2 < _NCH)
        def _():
            drain_meta(b2)
            mkidx(b2)

            @pl.when(k >= 2)
            def _():
                drain_scatter(b2)
            fire_gather(b2)

    def process(k, b):
        rows = rowss[b]
        pltpu.make_async_copy(h_h.at[idxs[b]], rowss[b], semgs[b]).wait()
        base = k * _CHUNK

        dnums = lax.GatherDimensionNumbers(offset_dims=(),
                                           collapsed_slice_dims=(0,),
                                           start_index_map=(0,))

        def scale(g, _):
            nv = normb[pl.ds(base + g * 16, 16)]
            for r16 in range(16):
                nb = lax.gather(nv, jnp.full((16, 1), r16, i32), dnums, (1,),
                                mode=lax.GatherScatterMode.PROMISE_IN_BOUNDS)
                r = g * 16 + r16
                for j in range(HH // 16):
                    rows[r, pl.ds(j * 16, 16)] = rows[r, pl.ds(j * 16, 16)] * nb
            return 0
        lax.fori_loop(0, _CHUNK // 16, scale, 0)

        def cpdst(j, _):
            sl = pl.ds(j * 16, 16)
            dscs[b][sl] = dstcs[b][sl]
            return 0
        lax.fori_loop(0, _CHUNK // 16, cpdst, 0, unroll=True)
        pltpu.async_copy(rows, acc.at[dscs[b]], semss[b], add=True)

    def whole_step(k, b, b2, bm):
        step1(k, b2)
        process(k, b)

        @pl.when(k + 4 < _NCH)
        def _():
            fire_meta(k + 4, bm)

    fire_meta(0, 0)
    fire_meta(1, 1)
    fire_meta(2, 2)
    fire_meta(3, 3)
    drain_meta(0)
    mkidx(0)
    fire_gather(0)
    drain_meta(1)
    mkidx(1)
    fire_gather(1)

    def body(kk, _):
        for j in range(_NB):
            k = kk * _NB + j
            whole_step(k, j, (j + 2) % _NB, j)
        return 0

    lax.fori_loop(0, _NCH // _NB, body, 0)
    drain_scatter((_NCH - 4) % _NB)
    drain_scatter((_NCH - 3) % _NB)
    drain_scatter((_NCH - 2) % _NB)
    drain_scatter((_NCH - 1) % _NB)
    plsc.subcore_barrier()
    pltpu.sync_copy(acc.at[pl.ds(row0, _RPT)], msg_h.at[pl.ds(c * NPAD + row0, _RPT)])


@functools.cache
def _conv():
    return pl.kernel(
    _conv_body,
    out_type=jax.ShapeDtypeStruct((2 * NPAD, HH), f32),
    mesh=_mesh(),
    compiler_params=_SC_PARAMS,
    scratch_types=[pltpu.VMEM_SHARED((NPAD, HH), f32),
                   pltpu.VMEM((_EPT,), f32)]
                  + [pltpu.VMEM((_CHUNK,), i32)] * 16
                  + [pltpu.VMEM((_CHUNK, HH), f32)] * 4
                  + [pltpu.SemaphoreType.DMA] * 12,
    )


def _matmul_init_body(x_ref, w_ref, b_ref, dis_ref, h_ref, init_ref):
    x = x_ref[...]
    h = jnp.dot(x, w_ref[...], preferred_element_type=f32)
    d2 = dis_ref[...] * dis_ref[...]
    init = h * d2 + b_ref[...]
    h_ref[0] = h[:, :HH]
    h_ref[1] = h[:, HH:]
    init_ref[0] = init[:, :HH]
    init_ref[1] = init[:, HH:]


def _matmul_init(x, w, b, dis):
    return pl.pallas_call(
        _matmul_init_body,
        out_shape=[jax.ShapeDtypeStruct((2, NPAD, HH), f32),
                   jax.ShapeDtypeStruct((2, NPAD, HH), f32)],
    )(x, w, b.reshape(1, D), dis.reshape(NPAD, 1))


def _bn_half(m, g, be, mask):
    mu = jnp.sum(m * mask, axis=0, keepdims=True) * (1.0 / N)
    dmu = m - mu
    var = jnp.sum(dmu * dmu * mask, axis=0, keepdims=True) * (1.0 / N)
    return g * dmu * jax.lax.rsqrt(var + 1e-5) + be


def _bn_relu_nores_body(msg_ref, g_ref, be_ref, y_ref):
    mask = (lax.broadcasted_iota(i32, (NPAD, 1), 0) < N).astype(f32)
    for half in range(2):
        m = msg_ref[half]
        g = g_ref[0:1, half * HH:half * HH + HH]
        be = be_ref[0:1, half * HH:half * HH + HH]
        y = _bn_half(m, g, be, mask)
        y_ref[:, half * HH:half * HH + HH] = jnp.maximum(y, 0.0)


def _bn_relu_res_body(msg_ref, g_ref, be_ref, res_ref, y_ref):
    mask = (lax.broadcasted_iota(i32, (NPAD, 1), 0) < N).astype(f32)
    for half in range(2):
        m = msg_ref[half]
        g = g_ref[0:1, half * HH:half * HH + HH]
        be = be_ref[0:1, half * HH:half * HH + HH]
        y = _bn_half(m, g, be, mask)
        y = y + res_ref[:, half * HH:half * HH + HH]
        y_ref[:, half * HH:half * HH + HH] = jnp.maximum(y, 0.0)


def _bn_relu(msg2, g, be, res=None):
    args = [msg2, g.reshape(1, D), be.reshape(1, D)]
    body = _bn_relu_nores_body
    if res is not None:
        args.append(res)
        body = _bn_relu_res_body
    return pl.pallas_call(
        body,
        out_shape=jax.ShapeDtypeStruct((NPAD, D), f32),
    )(*args)


def _final_body(msg_ref, g_ref, be_ref, res_ref, batch_ref,
                lw0_ref, lb0_ref, lw1_ref, lb1_ref, out_ref):
    mask = (lax.broadcasted_iota(i32, (NPAD, 1), 0) < N).astype(f32)
    halves = []
    for half in range(2):
        m = msg_ref[half]
        g = g_ref[0:1, half * HH:half * HH + HH]
        be = be_ref[0:1, half * HH:half * HH + HH]
        y = _bn_half(m, g, be, mask)
        y = y + res_ref[:, half * HH:half * HH + HH]
        halves.append(jnp.maximum(y, 0.0))
    y4 = jnp.concatenate(halves, axis=1)
    seg = lax.broadcasted_iota(i32, (NG, NPAD), 0)
    onehot = (seg == batch_ref[...]).astype(f32)
    pooled = jnp.dot(onehot, y4, preferred_element_type=f32,
                     precision=jax.lax.Precision.HIGHEST)
    cnt = jnp.sum(onehot, axis=1, keepdims=True)
    mean = pooled / jnp.maximum(cnt, 1.0)
    z = jnp.maximum(jnp.dot(mean, lw0_ref[...], preferred_element_type=f32)
                    + lb0_ref[...], 0.0)
    out_ref[...] = (jnp.dot(z, lw1_ref[...], preferred_element_type=f32)
                    + lb1_ref[...])


def _final(msg2, g, be, res, batch2, lw0, lb0, lw1, lb1):
    return pl.pallas_call(
        _final_body,
        out_shape=jax.ShapeDtypeStruct((NG, lw1.shape[1]), f32),
    )(msg2, g.reshape(1, D), be.reshape(1, D), res, batch2,
      lw0, lb0.reshape(1, -1), lw1, lb1.reshape(1, -1))


def kernel(x, edge_index, pos, batch, inv_sigma,
           W1_0, b1_0, g1_0, be1_0, W2_0, b2_0, g2_0, be2_0,
           W1_1, b1_1, g1_1, be1_1, W2_1, b2_1, g2_1, be2_1,
           lW0, lb0, lW1, lb1):
    npad = NPAD - N
    epad = EPAD - E
    xp = jnp.pad(x, ((0, npad), (0, 0)))
    posp = jnp.pad(pos, ((0, npad), (0, 0)))
    px = posp[:, 0] + 0.0
    py = posp[:, 1] + 0.0
    pz = posp[:, 2] + 0.0
    padi = (N + (jnp.arange(epad, dtype=i32) % npad)).astype(i32)
    srcp = jnp.concatenate([edge_index[0].astype(i32), padi])
    dstp = jnp.concatenate([edge_index[1].astype(i32), padi])
    batchp = jnp.pad(batch.astype(i32), (0, npad), constant_values=NG)
    batch2 = batchp.reshape(1, NPAD)
    sigb = jnp.broadcast_to(inv_sigma.astype(f32), (16,))

    ew, degp = _prep1()(srcp, dstp, px, py, pz, sigb)
    dis = _prep2()(degp)
    norm = _prep3()(srcp, dstp, ew, dis)

    def conv(h_in, w, b):
        h2, init2 = _matmul_init(h_in, w, b, dis)
        m = _conv()(h2.reshape(2 * NPAD, HH), init2.reshape(2 * NPAD, HH),
                    srcp, dstp, norm)
        return m.reshape(2, NPAD, HH)

    m = conv(xp, W1_0, b1_0)
    y = _bn_relu(m, g1_0, be1_0)
    m = conv(y, W2_0, b2_0)
    y2 = _bn_relu(m, g2_0, be2_0, res=xp)
    m = conv(y2, W1_1, b1_1)
    y = _bn_relu(m, g1_1, be1_1)
    m = conv(y, W2_1, b2_1)
    return _final(m, g2_1, be2_1, y2, batch2, lW0, lb0, lW1, lb1)
